# Initial kernel scaffold; baseline (speedup 1.0000x reference)
import jax
import jax.numpy as jnp
from jax.experimental import pallas as pl


def kernel(x, y, centers):
    raise NotImplementedError("write your pallas kernel here")


# speedup vs baseline: 1.2921x; 1.2921x over previous
import functools

import jax
import jax.numpy as jnp
from jax.experimental import pallas as pl
from jax.experimental.pallas import tpu as pltpu

_B = 16384
_C = 1000
_N = 1000
_BM = 1024


def _mse_kernel(x_ref, y_ref, centers_ref, out_ref):
    i = pl.program_id(0)

    y_blk = y_ref[...]
    classes = jax.lax.broadcasted_iota(jnp.int32, (_BM, _C), 1)
    onehot = (classes == y_blk).astype(jnp.float32)
    gathered = jnp.dot(onehot, centers_ref[...],
                       preferred_element_type=jnp.float32)
    d = x_ref[...] - gathered
    part = jnp.sum(d * d)

    @pl.when(i == 0)
    def _init():
        out_ref[0, 0] = 0.0

    out_ref[0, 0] += part


def kernel(x, y, centers):
    y2d = y.astype(jnp.int32).reshape(_B, 1)
    grid = _B // _BM
    total = pl.pallas_call(
        _mse_kernel,
        grid=(grid,),
        in_specs=[
            pl.BlockSpec((_BM, _N), lambda i: (i, 0)),
            pl.BlockSpec((_BM, 1), lambda i: (i, 0)),
            pl.BlockSpec((_C, _N), lambda i: (0, 0)),
        ],
        out_specs=pl.BlockSpec((1, 1), lambda i: (0, 0),
                               memory_space=pltpu.SMEM),
        out_shape=jax.ShapeDtypeStruct((1, 1), jnp.float32),
    )(x, y2d, centers)
    return (total[0, 0] / (_B * _N)).astype(jnp.float32)
